# baseline (speedup 1.0000x reference)
import functools

import jax
import jax.numpy as jnp
from jax import lax
from jax.experimental import pallas as pl
from jax.experimental.pallas import tpu as pltpu
from jax.experimental.pallas import tpu_sc as plsc

_V = 1000000
_D = 16
_NUM_ROWS = 16384 * 26
_NC = 2
_NS = 16
_NW = _NC * _NS

_mesh = plsc.VectorSubcoreMesh(core_axis_name="c", subcore_axis_name="s")

_VFULL = 999936
_W = 3840
_NBLK = (_VFULL + _W - 1) // _W
_WT = _VFULL - (_NBLK - 1) * _W
_KMAX = (_NBLK + _NW - 1) // _NW
_TAIL = _V - _VFULL


@functools.partial(
    pl.kernel,
    mesh=_mesh,
    out_type=jax.ShapeDtypeStruct((_V * _D,), jnp.float32),
    scratch_types=[
        pltpu.VMEM((_D, _W), jnp.float32),
        pltpu.VMEM((_W * _D,), jnp.float32),
    ],
    compiler_params=pltpu.CompilerParams(
        use_tc_tiling_on_sc=True, needs_layout_passes=False
    ),
)
def _regroup(tt, tail, out, buf_in, buf_out):
    wid = lax.axis_index("s") * _NC + lax.axis_index("c")
    dvec = lax.iota(jnp.int32, 16)
    zero = dvec * 0

    def do_block(c, w):
        pltpu.sync_copy(tt.at[:, pl.ds(c, w)], buf_in.at[:, pl.ds(0, w)])

        def emit(m, _):
            base = m * 8
            for v in range(8):
                val = plsc.load_gather(buf_in, [dvec, zero + (base + v)])
                buf_out[pl.ds(m * 128 + v * 16, 16)] = val
            return _

        lax.fori_loop(0, w // 8, emit, 0)
        pltpu.sync_copy(buf_out.at[pl.ds(0, w * _D)], out.at[pl.ds(c * _D, w * _D)])

    for k in range(_KMAX):
        blk = wid + k * _NW
        if k < _KMAX - 1:
            do_block(blk * _W, _W)
        else:
            @pl.when(blk < _NBLK - 1)
            def _():
                do_block(blk * _W, _W)

            @pl.when(blk == _NBLK - 1)
            def _():
                do_block((_NBLK - 1) * _W, _WT)

            @pl.when(blk == _NBLK)
            def _():
                pltpu.sync_copy(tail, buf_out.at[pl.ds(0, _TAIL * _D)])
                pltpu.sync_copy(
                    buf_out.at[pl.ds(0, _TAIL * _D)],
                    out.at[pl.ds(_VFULL * _D, _TAIL * _D)],
                )


_B_PER_W = _NUM_ROWS // _NW
_CHUNK = 1664
_NCHUNK = _B_PER_W // _CHUNK
_NBUF = 4


@functools.partial(
    pl.kernel,
    mesh=_mesh,
    out_type=jax.ShapeDtypeStruct((_NUM_ROWS, _D), jnp.float32),
    scratch_types=[
        pltpu.VMEM((_B_PER_W,), jnp.int32),
        [pltpu.VMEM((_CHUNK, _D), jnp.float32) for _ in range(_NBUF)],
        [pltpu.SemaphoreType.DMA for _ in range(_NBUF)],
        [pltpu.SemaphoreType.DMA for _ in range(_NBUF)],
    ],
    compiler_params=pltpu.CompilerParams(use_tc_tiling_on_sc=False),
)
def _gather_rows(idx_hbm, table_hbm, out_hbm, idx_v, rows, g_sems, o_sems):
    wid = lax.axis_index("s") * _NC + lax.axis_index("c")
    base = wid * _B_PER_W

    pltpu.sync_copy(idx_hbm.at[pl.ds(base, _B_PER_W)], idx_v)

    def start_gather(i, b):
        idx_slice = idx_v.at[pl.ds(i * _CHUNK, _CHUNK)]
        return pltpu.async_copy(table_hbm.at[idx_slice], rows[b], g_sems[b])

    gather_dma = [None] * _NCHUNK
    out_dma = [None] * _NCHUNK
    for b in range(_NBUF):
        gather_dma[b] = start_gather(b, b)

    for i in range(_NCHUNK):
        b = i % _NBUF
        gather_dma[i].wait()
        out_dma[i] = pltpu.async_copy(
            rows[b], out_hbm.at[pl.ds(base + i * _CHUNK, _CHUNK)], o_sems[b]
        )
        nxt = i + _NBUF
        if nxt < _NCHUNK:
            out_dma[i].wait()
            gather_dma[nxt] = start_gather(nxt, b)

    for i in range(max(0, _NCHUNK - _NBUF), _NCHUNK):
        out_dma[i].wait()


def kernel(x, table):
    tail = table[_VFULL:].reshape(-1)
    flat_table = _regroup(table.T, tail)
    out = _gather_rows(x.reshape(-1), flat_table.reshape(_V, _D))
    return out.reshape(x.shape[0], -1)


# speedup vs baseline: 1.4582x; 1.4582x over previous
import functools

import jax
import jax.numpy as jnp
from jax import lax
from jax.experimental import pallas as pl
from jax.experimental.pallas import tpu as pltpu
from jax.experimental.pallas import tpu_sc as plsc

_V = 1000000
_D = 16
_NUM_ROWS = 16384 * 26
_NC = 2
_NS = 16
_NW = _NC * _NS

_mesh = plsc.VectorSubcoreMesh(core_axis_name="c", subcore_axis_name="s")

_VFULL = 999936
_W = 3840
_NBLK = (_VFULL + _W - 1) // _W
_WT = _VFULL - (_NBLK - 1) * _W
_KMAX = (_NBLK + _NW - 1) // _NW
_TAIL = _V - _VFULL


@functools.partial(
    pl.kernel,
    mesh=_mesh,
    out_type=jax.ShapeDtypeStruct((_V * _D,), jnp.float32),
    scratch_types=[
        pltpu.VMEM((_D, _W), jnp.float32),
        pltpu.VMEM((_W * _D,), jnp.float32),
    ],
    compiler_params=pltpu.CompilerParams(
        use_tc_tiling_on_sc=True, needs_layout_passes=False
    ),
)
def _regroup(tt, tail, out, buf_in, buf_out):
    wid = lax.axis_index("s") * _NC + lax.axis_index("c")
    dvec = lax.iota(jnp.int32, 16)
    zero = dvec * 0

    def do_block(c, w):
        pltpu.sync_copy(tt.at[:, pl.ds(c, w)], buf_in.at[:, pl.ds(0, w)])

        @plsc.parallel_loop(0, w // 8, unroll=4)
        def emit(m):
            base = m * 8
            for v in range(8):
                val = plsc.load_gather(buf_in, [dvec, zero + (base + v)])
                buf_out[pl.ds(m * 128 + v * 16, 16)] = val
        pltpu.sync_copy(buf_out.at[pl.ds(0, w * _D)], out.at[pl.ds(c * _D, w * _D)])

    for k in range(_KMAX):
        blk = wid + k * _NW
        if k < _KMAX - 1:
            do_block(blk * _W, _W)
        else:
            @pl.when(blk < _NBLK - 1)
            def _():
                do_block(blk * _W, _W)

            @pl.when(blk == _NBLK - 1)
            def _():
                do_block((_NBLK - 1) * _W, _WT)

            @pl.when(blk == _NBLK)
            def _():
                pltpu.sync_copy(tail, buf_out.at[pl.ds(0, _TAIL * _D)])
                pltpu.sync_copy(
                    buf_out.at[pl.ds(0, _TAIL * _D)],
                    out.at[pl.ds(_VFULL * _D, _TAIL * _D)],
                )


_B_PER_W = _NUM_ROWS // _NW
_CHUNK = 1664
_NCHUNK = _B_PER_W // _CHUNK
_NBUF = 4


@functools.partial(
    pl.kernel,
    mesh=_mesh,
    out_type=jax.ShapeDtypeStruct((_NUM_ROWS, _D), jnp.float32),
    scratch_types=[
        pltpu.VMEM((_B_PER_W,), jnp.int32),
        [pltpu.VMEM((_CHUNK, _D), jnp.float32) for _ in range(_NBUF)],
        [pltpu.SemaphoreType.DMA for _ in range(_NBUF)],
        [pltpu.SemaphoreType.DMA for _ in range(_NBUF)],
    ],
    compiler_params=pltpu.CompilerParams(use_tc_tiling_on_sc=False),
)
def _gather_rows(idx_hbm, table_hbm, out_hbm, idx_v, rows, g_sems, o_sems):
    wid = lax.axis_index("s") * _NC + lax.axis_index("c")
    base = wid * _B_PER_W

    pltpu.sync_copy(idx_hbm.at[pl.ds(base, _B_PER_W)], idx_v)

    def start_gather(i, b):
        idx_slice = idx_v.at[pl.ds(i * _CHUNK, _CHUNK)]
        return pltpu.async_copy(table_hbm.at[idx_slice], rows[b], g_sems[b])

    gather_dma = [None] * _NCHUNK
    out_dma = [None] * _NCHUNK
    for b in range(_NBUF):
        gather_dma[b] = start_gather(b, b)

    for i in range(_NCHUNK):
        b = i % _NBUF
        gather_dma[i].wait()
        out_dma[i] = pltpu.async_copy(
            rows[b], out_hbm.at[pl.ds(base + i * _CHUNK, _CHUNK)], o_sems[b]
        )
        nxt = i + _NBUF
        if nxt < _NCHUNK:
            out_dma[i].wait()
            gather_dma[nxt] = start_gather(nxt, b)

    for i in range(max(0, _NCHUNK - _NBUF), _NCHUNK):
        out_dma[i].wait()


def kernel(x, table):
    tail = table[_VFULL:].reshape(-1)
    flat_table = _regroup(table.T, tail)
    out = _gather_rows(x.reshape(-1), flat_table.reshape(_V, _D))
    return out.reshape(x.shape[0], -1)


# speedup vs baseline: 1.6946x; 1.1621x over previous
import functools

import jax
import jax.numpy as jnp
from jax import lax
from jax.experimental import pallas as pl
from jax.experimental.pallas import tpu as pltpu
from jax.experimental.pallas import tpu_sc as plsc

_V = 1000000
_D = 16
_NUM_ROWS = 16384 * 26
_NC = 2
_NS = 16
_NW = _NC * _NS

_mesh = plsc.VectorSubcoreMesh(core_axis_name="c", subcore_axis_name="s")

_VFULL = 999936
_W = 1920
_NBLK = (_VFULL + _W - 1) // _W
_WT = _VFULL - (_NBLK - 1) * _W
_KFULL = 16
_TAIL = _V - _VFULL


@functools.partial(
    pl.kernel,
    mesh=_mesh,
    out_type=jax.ShapeDtypeStruct((_V * _D,), jnp.float32),
    scratch_types=[
        [pltpu.VMEM((_D, _W), jnp.float32) for _ in range(2)],
        [pltpu.VMEM((_W * _D,), jnp.float32) for _ in range(2)],
        [pltpu.SemaphoreType.DMA for _ in range(2)],
        [pltpu.SemaphoreType.DMA for _ in range(2)],
    ],
    compiler_params=pltpu.CompilerParams(
        use_tc_tiling_on_sc=True, needs_layout_passes=False
    ),
)
def _regroup(tt, tail, out, bin_, bout, s_in, s_out):
    wid = lax.axis_index("s") * _NC + lax.axis_index("c")
    dvec = lax.iota(jnp.int32, 16)
    zero = dvec * 0

    def start_in(k, p):
        c = (wid + k * _NW) * _W
        return pltpu.async_copy(tt.at[:, pl.ds(c, _W)], bin_[p], s_in[p])

    def shuffle(p, w):
        @plsc.parallel_loop(0, w // 8, unroll=4)
        def emit(m):
            base = m * 8
            for v in range(8):
                val = plsc.load_gather(bin_[p], [dvec, zero + (base + v)])
                bout[p][pl.ds(m * 128 + v * 16, 16)] = val

    def start_out(k, p, w):
        c = (wid + k * _NW) * _W
        return pltpu.async_copy(
            bout[p].at[pl.ds(0, w * _D)], out.at[pl.ds(c * _D, w * _D)], s_out[p]
        )

    d_in = [None] * _KFULL
    d_out = [None] * _KFULL
    d_in[0] = start_in(0, 0)
    for k in range(_KFULL):
        p = k % 2
        if k + 1 < _KFULL:
            d_in[k + 1] = start_in(k + 1, 1 - p)
        d_in[k].wait()
        if k >= 2:
            d_out[k - 2].wait()
        shuffle(p, _W)
        d_out[k] = start_out(k, p, _W)
    d_out[_KFULL - 2].wait()
    d_out[_KFULL - 1].wait()

    blk = wid + _KFULL * _NW

    @pl.when(blk < _NBLK - 1)
    def _():
        start_in(_KFULL, 0).wait()
        shuffle(0, _W)
        start_out(_KFULL, 0, _W).wait()

    @pl.when(blk == _NBLK - 1)
    def _():
        c = (_NBLK - 1) * _W
        pltpu.async_copy(tt.at[:, pl.ds(c, _WT)], bin_[0].at[:, pl.ds(0, _WT)],
                         s_in[0]).wait()
        shuffle(0, _WT)
        pltpu.async_copy(bout[0].at[pl.ds(0, _WT * _D)],
                         out.at[pl.ds(c * _D, _WT * _D)], s_out[0]).wait()

    @pl.when(blk == _NBLK)
    def _():
        pltpu.sync_copy(tail, bout[0].at[pl.ds(0, _TAIL * _D)])
        pltpu.sync_copy(
            bout[0].at[pl.ds(0, _TAIL * _D)],
            out.at[pl.ds(_VFULL * _D, _TAIL * _D)],
        )


_B_PER_W = _NUM_ROWS // _NW
_CHUNK = 1664
_NCHUNK = _B_PER_W // _CHUNK
_NBUF = 4


@functools.partial(
    pl.kernel,
    mesh=_mesh,
    out_type=jax.ShapeDtypeStruct((_NUM_ROWS, _D), jnp.float32),
    scratch_types=[
        pltpu.VMEM((_B_PER_W,), jnp.int32),
        [pltpu.VMEM((_CHUNK, _D), jnp.float32) for _ in range(_NBUF)],
        [pltpu.SemaphoreType.DMA for _ in range(_NBUF)],
        [pltpu.SemaphoreType.DMA for _ in range(_NBUF)],
    ],
    compiler_params=pltpu.CompilerParams(use_tc_tiling_on_sc=False),
)
def _gather_rows(idx_hbm, table_hbm, out_hbm, idx_v, rows, g_sems, o_sems):
    wid = lax.axis_index("s") * _NC + lax.axis_index("c")
    base = wid * _B_PER_W

    pltpu.sync_copy(idx_hbm.at[pl.ds(base, _B_PER_W)], idx_v)

    def start_gather(i, b):
        idx_slice = idx_v.at[pl.ds(i * _CHUNK, _CHUNK)]
        return pltpu.async_copy(table_hbm.at[idx_slice], rows[b], g_sems[b])

    gather_dma = [None] * _NCHUNK
    out_dma = [None] * _NCHUNK
    for b in range(_NBUF):
        gather_dma[b] = start_gather(b, b)

    for i in range(_NCHUNK):
        b = i % _NBUF
        gather_dma[i].wait()
        out_dma[i] = pltpu.async_copy(
            rows[b], out_hbm.at[pl.ds(base + i * _CHUNK, _CHUNK)], o_sems[b]
        )
        nxt = i + _NBUF
        if nxt < _NCHUNK:
            out_dma[i].wait()
            gather_dma[nxt] = start_gather(nxt, b)

    for i in range(max(0, _NCHUNK - _NBUF), _NCHUNK):
        out_dma[i].wait()


def kernel(x, table):
    tail = table[_VFULL:].reshape(-1)
    flat_table = _regroup(table.T, tail)
    out = _gather_rows(x.reshape(-1), flat_table.reshape(_V, _D))
    return out.reshape(x.shape[0], -1)


# speedup vs baseline: 3.3054x; 1.9506x over previous
import functools

import jax
import jax.numpy as jnp
from jax import lax
from jax.experimental import pallas as pl
from jax.experimental.pallas import tpu as pltpu
from jax.experimental.pallas import tpu_sc as plsc

_V = 1000000
_D = 16
_NUM_ROWS = 16384 * 26
_NC = 2
_NS = 16
_NW = _NC * _NS

_mesh = plsc.VectorSubcoreMesh(core_axis_name="c", subcore_axis_name="s")

_VFULL = 999936
_W = 1920
_NBLK = (_VFULL + _W - 1) // _W
_WT = _VFULL - (_NBLK - 1) * _W
_KFULL = 16
_TAIL = _V - _VFULL


@functools.partial(
    pl.kernel,
    mesh=_mesh,
    out_type=jax.ShapeDtypeStruct((_V * _D,), jnp.float32),
    scratch_types=[
        [pltpu.VMEM((_D, _W), jnp.float32) for _ in range(2)],
        [pltpu.VMEM((_W * _D,), jnp.float32) for _ in range(2)],
        [pltpu.SemaphoreType.DMA for _ in range(2)],
        [pltpu.SemaphoreType.DMA for _ in range(2)],
    ],
    compiler_params=pltpu.CompilerParams(
        use_tc_tiling_on_sc=True, needs_layout_passes=False
    ),
)
def _regroup(tt, tail, out, bin_, bout, s_in, s_out):
    wid = lax.axis_index("s") * _NC + lax.axis_index("c")
    dvec = lax.iota(jnp.int32, 16)
    zero = dvec * 0

    def start_in(k, p):
        c = (wid + k * _NW) * _W
        return pltpu.async_copy(tt.at[:, pl.ds(c, _W)], bin_[p], s_in[p])

    idx16 = dvec * _D

    def shuffle(p, w):
        @plsc.parallel_loop(0, w // 16, unroll=2)
        def emit(m):
            l0 = m * 16
            for d in range(_D):
                val = bin_[p][d, pl.ds(l0, 16)]
                plsc.store_scatter(bout[p], [idx16 + (l0 * _D + d)], val)

    def start_out(k, p, w):
        c = (wid + k * _NW) * _W
        return pltpu.async_copy(
            bout[p].at[pl.ds(0, w * _D)], out.at[pl.ds(c * _D, w * _D)], s_out[p]
        )

    d_in = [None] * _KFULL
    d_out = [None] * _KFULL
    d_in[0] = start_in(0, 0)
    for k in range(_KFULL):
        p = k % 2
        if k + 1 < _KFULL:
            d_in[k + 1] = start_in(k + 1, 1 - p)
        d_in[k].wait()
        if k >= 2:
            d_out[k - 2].wait()
        shuffle(p, _W)
        d_out[k] = start_out(k, p, _W)
    d_out[_KFULL - 2].wait()
    d_out[_KFULL - 1].wait()

    blk = wid + _KFULL * _NW

    @pl.when(blk < _NBLK - 1)
    def _():
        start_in(_KFULL, 0).wait()
        shuffle(0, _W)
        start_out(_KFULL, 0, _W).wait()

    @pl.when(blk == _NBLK - 1)
    def _():
        c = (_NBLK - 1) * _W
        pltpu.async_copy(tt.at[:, pl.ds(c, _WT)], bin_[0].at[:, pl.ds(0, _WT)],
                         s_in[0]).wait()
        shuffle(0, _WT)
        pltpu.async_copy(bout[0].at[pl.ds(0, _WT * _D)],
                         out.at[pl.ds(c * _D, _WT * _D)], s_out[0]).wait()

    @pl.when(blk == _NBLK)
    def _():
        pltpu.sync_copy(tail, bout[0].at[pl.ds(0, _TAIL * _D)])
        pltpu.sync_copy(
            bout[0].at[pl.ds(0, _TAIL * _D)],
            out.at[pl.ds(_VFULL * _D, _TAIL * _D)],
        )


_B_PER_W = _NUM_ROWS // _NW
_CHUNK = 1664
_NCHUNK = _B_PER_W // _CHUNK
_NBUF = 4


@functools.partial(
    pl.kernel,
    mesh=_mesh,
    out_type=jax.ShapeDtypeStruct((_NUM_ROWS, _D), jnp.float32),
    scratch_types=[
        pltpu.VMEM((_B_PER_W,), jnp.int32),
        [pltpu.VMEM((_CHUNK, _D), jnp.float32) for _ in range(_NBUF)],
        [pltpu.SemaphoreType.DMA for _ in range(_NBUF)],
        [pltpu.SemaphoreType.DMA for _ in range(_NBUF)],
    ],
    compiler_params=pltpu.CompilerParams(use_tc_tiling_on_sc=False),
)
def _gather_rows(idx_hbm, table_hbm, out_hbm, idx_v, rows, g_sems, o_sems):
    wid = lax.axis_index("s") * _NC + lax.axis_index("c")
    base = wid * _B_PER_W

    pltpu.sync_copy(idx_hbm.at[pl.ds(base, _B_PER_W)], idx_v)

    def start_gather(i, b):
        idx_slice = idx_v.at[pl.ds(i * _CHUNK, _CHUNK)]
        return pltpu.async_copy(table_hbm.at[idx_slice], rows[b], g_sems[b])

    gather_dma = [None] * _NCHUNK
    out_dma = [None] * _NCHUNK
    for b in range(_NBUF):
        gather_dma[b] = start_gather(b, b)

    for i in range(_NCHUNK):
        b = i % _NBUF
        gather_dma[i].wait()
        out_dma[i] = pltpu.async_copy(
            rows[b], out_hbm.at[pl.ds(base + i * _CHUNK, _CHUNK)], o_sems[b]
        )
        nxt = i + _NBUF
        if nxt < _NCHUNK:
            out_dma[i].wait()
            gather_dma[nxt] = start_gather(nxt, b)

    for i in range(max(0, _NCHUNK - _NBUF), _NCHUNK):
        out_dma[i].wait()


def kernel(x, table):
    tail = table[_VFULL:].reshape(-1)
    flat_table = _regroup(table.T, tail)
    out = _gather_rows(x.reshape(-1), flat_table.reshape(_V, _D))
    return out.reshape(x.shape[0], -1)
